# Initial kernel scaffold; baseline (speedup 1.0000x reference)
import jax
import jax.numpy as jnp
from jax.experimental import pallas as pl


def kernel(x, W, b):
    raise NotImplementedError("write your pallas kernel here")


# speedup vs baseline: 2.4851x; 2.4851x over previous
import functools

import jax
import jax.numpy as jnp
from jax import lax
from jax.experimental import pallas as pl
from jax.experimental.pallas import tpu as pltpu
from jax.experimental.pallas import tpu_sc as plsc

B, A, O, K, D = 1024, 8, 64, 32, 128
N = B * A * O
NC, NS = 2, 16
NW = NC * NS
PER_W = N // NW
CHUNK = 128
NCH = PER_W // CHUNK


def _table_body(w_ref, b_ref, t_ref):
    t_ref[...] = w_ref[...] + b_ref[...]


def _fused_table(W, b):
    return pl.pallas_call(
        _table_body,
        out_shape=jax.ShapeDtypeStruct((K, D), jnp.float32),
    )(W, b.reshape(1, D))


_mesh = plsc.VectorSubcoreMesh(core_axis_name="c", subcore_axis_name="s")


@functools.partial(
    pl.kernel,
    mesh=_mesh,
    out_type=jax.ShapeDtypeStruct((N, D), jnp.float32),
    scratch_types=[
        pltpu.VMEM((NCH, CHUNK), jnp.int32),
        pltpu.VMEM((CHUNK, D), jnp.float32),
        pltpu.SemaphoreType.DMA,
    ],
)
def _sc_gather(table_hbm, idx_hbm, out_hbm, idx_v, rows_v, sem):
    wid = lax.axis_index("s") * NC + lax.axis_index("c")
    pltpu.sync_copy(idx_hbm.at[wid], idx_v)

    def body(j, carry):
        pltpu.async_copy(table_hbm.at[idx_v.at[j]], rows_v, sem).wait()
        pltpu.sync_copy(rows_v, out_hbm.at[pl.ds(wid * PER_W + j * CHUNK, CHUNK)])
        return carry

    lax.fori_loop(0, NCH, body, 0)


def kernel(x, W, b):
    table = _fused_table(W, b)
    idx = x.astype(jnp.int32).reshape(NW, NCH, CHUNK)
    y = _sc_gather(table, idx)
    return y.reshape(B, A, O, D)


# speedup vs baseline: 2.5083x; 1.0094x over previous
import functools

import jax
import jax.numpy as jnp
from jax import lax
from jax.experimental import pallas as pl
from jax.experimental.pallas import tpu as pltpu
from jax.experimental.pallas import tpu_sc as plsc

B, A, O, K, D = 1024, 8, 64, 32, 128
N = B * A * O
NC, NS = 2, 16
NW = NC * NS
PER_W = N // NW
CHUNK = 128
NCH = PER_W // CHUNK
NBUF = 4
NSUP = NCH // NBUF


def _table_body(w_ref, b_ref, t_ref):
    t_ref[...] = w_ref[...] + b_ref[...]


def _fused_table(W, b):
    return pl.pallas_call(
        _table_body,
        out_shape=jax.ShapeDtypeStruct((K, D), jnp.float32),
    )(W, b.reshape(1, D))


_mesh = plsc.VectorSubcoreMesh(core_axis_name="c", subcore_axis_name="s")


@functools.partial(
    pl.kernel,
    mesh=_mesh,
    out_type=jax.ShapeDtypeStruct((N, D), jnp.float32),
    scratch_types=[
        pltpu.VMEM((NCH, CHUNK), jnp.int32),
        pltpu.VMEM((NBUF, CHUNK, D), jnp.float32),
        pltpu.SemaphoreType.DMA((NBUF,)),
        pltpu.SemaphoreType.DMA((NBUF,)),
    ],
)
def _sc_gather(table_hbm, idx_hbm, out_hbm, idx_v, rows_v, sem_g, sem_s):
    wid = lax.axis_index("s") * NC + lax.axis_index("c")
    base = wid * PER_W
    pltpu.sync_copy(idx_hbm.at[wid], idx_v)

    def gather(j, b):
        return pltpu.make_async_copy(
            table_hbm.at[idx_v.at[j]], rows_v.at[b], sem_g.at[b])

    def scatter(j, b):
        return pltpu.make_async_copy(
            rows_v.at[b], out_hbm.at[pl.ds(base + j * CHUNK, CHUNK)],
            sem_s.at[b])

    for bb in range(NBUF):
        gather(bb, bb).start()
    for bb in range(NBUF):
        gather(bb, bb).wait()
        scatter(bb, bb).start()

    def body(g, carry):
        for bb in range(NBUF):
            j = g * NBUF + bb
            scatter(j - NBUF, bb).wait()
            gather(j, bb).start()
        for bb in range(NBUF):
            j = g * NBUF + bb
            gather(j, bb).wait()
            scatter(j, bb).start()
        return carry

    lax.fori_loop(1, NSUP, body, 0)

    for bb in range(NBUF):
        scatter(NCH - NBUF + bb, bb).wait()


def kernel(x, W, b):
    table = _fused_table(W, b)
    idx = x.astype(jnp.int32).reshape(NW, NCH, CHUNK)
    y = _sc_gather(table, idx)
    return y.reshape(B, A, O, D)
